# baseline (speedup 1.0000x reference)
import functools

import jax
import jax.numpy as jnp
from jax.experimental import pallas as pl
from jax.experimental.pallas import tpu as pltpu

N_WAY = 5
K_SHOT = 5
NBNN_TOPK = 3
TEMPERATURE = 0.1
G = 32
MSP = 128
QT = 25


def _mn4_kernel(qnat_ref, smat_ref, onehot_ref, out_ref, *, q, c, hw, m_s):
    nsp = N_WAY * MSP
    f32 = jnp.float32
    qnat = qnat_ref[0]
    smat = smat_ref[0]
    onehot = onehot_ref[0]

    qpad = jnp.concatenate(
        [qnat, jnp.zeros((q, c, G - hw), f32)], axis=2)
    qmat = jnp.swapaxes(qpad, 1, 2).reshape(q * G, c)

    raw = jnp.dot(qmat, smat, preferred_element_type=f32)
    qn2 = jnp.sum(qmat * qmat, axis=1, keepdims=True)
    sn2 = jnp.sum(smat * smat, axis=0, keepdims=True)
    rq = 1.0 / (jnp.sqrt(qn2) + 1e-12)
    rs = 1.0 / (jnp.sqrt(sn2) + 1e-12)

    iota_j = jax.lax.broadcasted_iota(jnp.int32, (1, 1, nsp), 2)
    lanevalid = (iota_j & (MSP - 1)) < m_s
    iota_jf = iota_j.astype(f32)
    iota_i = jax.lax.broadcasted_iota(jnp.int32, (1, G, 1), 1).astype(f32)
    valid = iota_i < float(hw)
    validf = valid.astype(f32)
    neg = f32(-3.0e38)

    sim = jnp.where(lanevalid, (raw * rq * rs).reshape(q, G, nsp), neg)

    cwm = jnp.max(sim, axis=2, keepdims=True)
    qn = jnp.min(jnp.where(sim == cwm, iota_jf, float(nsp)), axis=2,
                 keepdims=True)

    point = (qn == iota_jf) & valid
    cm = jnp.where(point, cwm + 1.0, 0.0)
    win_val = jnp.max(cm, axis=1, keepdims=True)
    win_idx = jnp.min(jnp.where(cm == win_val, iota_i, float(G)), axis=1,
                      keepdims=True)

    mask = jnp.max((point & (win_idx == iota_i)).astype(f32),
                   axis=2, keepdims=True)
    mask = mask * validf

    def top3_sum(chunk):
        m1 = jnp.max(chunk, axis=2, keepdims=True)
        eq1 = chunk == m1
        c1 = jnp.sum(eq1.astype(f32), axis=2, keepdims=True)
        x2 = jnp.where(eq1, neg, chunk)
        m2 = jnp.max(x2, axis=2, keepdims=True)
        eq2 = x2 == m2
        c2 = jnp.sum(eq2.astype(f32), axis=2, keepdims=True)
        m3 = jnp.max(jnp.where(eq2, neg, x2), axis=2, keepdims=True)
        second = jnp.where(c1 >= 2.0, m1, m2)
        third = jnp.where(c1 >= 3.0, m1,
                          jnp.where(c1 >= 2.0, m2,
                                    jnp.where(c2 >= 2.0, m2, m3)))
        return m1 + second + third

    qvs = []
    for n in range(N_WAY):
        val_n = top3_sum(sim[:, :, n * MSP:(n + 1) * MSP])
        qvs.append(jnp.sum(val_n * mask, axis=1, keepdims=True))
    logits = jnp.concatenate(qvs, axis=2) * f32(1.0 / (3.0 * TEMPERATURE))

    lm = jnp.max(logits, axis=2, keepdims=True)
    lse = lm + jnp.log(jnp.sum(jnp.exp(logits - lm), axis=2, keepdims=True))
    logp = logits - lse
    partial = jnp.zeros((1, 1, 1), f32) - jnp.sum(logp * onehot)

    @pl.when(pl.program_id(1) == 0)
    def _():
        out_ref[...] = jnp.zeros((1, 1, 1), f32)

    out_ref[...] += partial


def kernel(support_xf, support_y, query_xf, query_y):
    b, q, c, h, w = query_xf.shape
    hw = h * w
    m_s = K_SHOT * hw
    s_tot = N_WAY * K_SHOT
    nsp = N_WAY * MSP

    smat = jnp.transpose(support_xf.reshape(b, s_tot, c, hw),
                         (0, 2, 1, 3)).reshape(b, c, N_WAY, m_s)
    smat = jnp.pad(smat, ((0, 0), (0, 0), (0, 0), (0, MSP - m_s)))
    smat = smat.reshape(b, c, nsp)

    qnat = query_xf.reshape(b, q, c, hw)

    onehot = (query_y[..., None] == jnp.arange(N_WAY, dtype=query_y.dtype))
    onehot = onehot.astype(jnp.float32).reshape(b, q, 1, N_WAY)

    qt = QT
    assert q % qt == 0
    loss_sum = pl.pallas_call(
        functools.partial(_mn4_kernel, q=qt, c=c, hw=hw, m_s=m_s),
        grid=(b, q // qt),
        in_specs=[
            pl.BlockSpec((1, qt, c, hw), lambda i, j: (i, j, 0, 0)),
            pl.BlockSpec((1, c, nsp), lambda i, j: (i, 0, 0)),
            pl.BlockSpec((1, qt, 1, N_WAY), lambda i, j: (i, j, 0, 0)),
        ],
        out_specs=pl.BlockSpec((1, 1, 1), lambda i, j: (i, 0, 0)),
        out_shape=jax.ShapeDtypeStruct((b, 1, 1), jnp.float32),
        compiler_params=pltpu.CompilerParams(
            dimension_semantics=("parallel", "arbitrary")),
    )(qnat, smat, onehot)

    return jnp.sum(loss_sum) / (b * q)


# speedup vs baseline: 1.0427x; 1.0427x over previous
import functools

import jax
import jax.numpy as jnp
from jax.experimental import pallas as pl

N_WAY = 5
K_SHOT = 5
NBNN_TOPK = 3
TEMPERATURE = 0.1
G = 32
MSP = 128
QT = 25


def _mn4_kernel(qnat_ref, smat_ref, onehot_ref, out_ref, *, q, c, hw, m_s):
    nsp = N_WAY * MSP
    f32 = jnp.float32
    qnat = qnat_ref[0]
    smat = smat_ref[0]
    onehot = onehot_ref[0]

    qpad = jnp.concatenate(
        [qnat, jnp.zeros((q, c, G - hw), f32)], axis=2)
    qmat = jnp.swapaxes(qpad, 1, 2).reshape(q * G, c)

    raw = jnp.dot(qmat, smat, preferred_element_type=f32)
    qn2 = jnp.sum(qmat * qmat, axis=1, keepdims=True)
    sn2 = jnp.sum(smat * smat, axis=0, keepdims=True)
    rq = 1.0 / (jnp.sqrt(qn2) + 1e-12)
    rs = 1.0 / (jnp.sqrt(sn2) + 1e-12)

    iota_j = jax.lax.broadcasted_iota(jnp.int32, (1, 1, nsp), 2)
    lanevalid = (iota_j & (MSP - 1)) < m_s
    iota_jf = iota_j.astype(f32)
    iota_i = jax.lax.broadcasted_iota(jnp.int32, (1, G, 1), 1).astype(f32)
    valid = iota_i < float(hw)
    validf = valid.astype(f32)
    neg = f32(-3.0e38)

    sim = jnp.where(lanevalid, (raw * rq * rs).reshape(q, G, nsp), neg)

    iota_m = jax.lax.broadcasted_iota(jnp.int32, (1, 1, MSP), 2).astype(f32)

    def chunk_stats(chunk):
        m1 = jnp.max(chunk, axis=2, keepdims=True)
        eq1 = chunk == m1
        f1 = jnp.min(jnp.where(eq1, iota_m, float(MSP)), axis=2,
                     keepdims=True)
        c1 = jnp.sum(eq1.astype(f32), axis=2, keepdims=True)
        x2 = jnp.where(eq1, neg, chunk)
        m2 = jnp.max(x2, axis=2, keepdims=True)
        eq2 = x2 == m2
        c2 = jnp.sum(eq2.astype(f32), axis=2, keepdims=True)
        m3 = jnp.max(jnp.where(eq2, neg, x2), axis=2, keepdims=True)
        second = jnp.where(c1 >= 2.0, m1, m2)
        third = jnp.where(c1 >= 3.0, m1,
                          jnp.where(c1 >= 2.0, m2,
                                    jnp.where(c2 >= 2.0, m2, m3)))
        return m1, f1, m1 + second + third

    stats = [chunk_stats(sim[:, :, n * MSP:(n + 1) * MSP])
             for n in range(N_WAY)]

    cwm = stats[0][0]
    for n in range(1, N_WAY):
        cwm = jnp.maximum(cwm, stats[n][0])
    qn = jnp.full_like(cwm, float(N_WAY * MSP))
    for n in range(N_WAY):
        m1_n, f1_n, _ = stats[n]
        qn = jnp.minimum(qn, jnp.where(m1_n == cwm,
                                       f32(n * MSP) + f1_n, qn))

    qn_r = jnp.swapaxes(qn, 1, 2)
    cwm_r = jnp.swapaxes(cwm, 1, 2)
    iota_ir = jax.lax.broadcasted_iota(jnp.int32, (1, 1, G), 2).astype(f32)
    valid_r = iota_ir < float(hw)
    better = ((qn_r == qn) & valid_r
              & ((cwm_r > cwm)
                 | ((cwm_r == cwm) & (iota_ir < iota_i))))
    beaten = jnp.max(better.astype(f32), axis=2, keepdims=True)
    mask = (1.0 - beaten) * validf

    qvs = [jnp.sum(stats[n][2] * mask, axis=1, keepdims=True)
           for n in range(N_WAY)]
    logits = jnp.concatenate(qvs, axis=2) * f32(1.0 / (3.0 * TEMPERATURE))

    lm = jnp.max(logits, axis=2, keepdims=True)
    lse = lm + jnp.log(jnp.sum(jnp.exp(logits - lm), axis=2, keepdims=True))
    logp = logits - lse
    partial = jnp.zeros((1, 1), f32) - jnp.sum(logp * onehot)

    @pl.when((pl.program_id(0) == 0) & (pl.program_id(1) == 0))
    def _():
        out_ref[...] = jnp.zeros((1, 1), f32)

    out_ref[...] += partial


def kernel(support_xf, support_y, query_xf, query_y):
    b, q, c, h, w = query_xf.shape
    hw = h * w
    m_s = K_SHOT * hw
    s_tot = N_WAY * K_SHOT
    nsp = N_WAY * MSP

    smat = jnp.transpose(support_xf.reshape(b, s_tot, c, hw),
                         (0, 2, 1, 3)).reshape(b, c, N_WAY, m_s)
    smat = jnp.pad(smat, ((0, 0), (0, 0), (0, 0), (0, MSP - m_s)))
    smat = smat.reshape(b, c, nsp)

    qnat = query_xf.reshape(b, q, c, hw)

    onehot = (query_y[..., None] == jnp.arange(N_WAY, dtype=query_y.dtype))
    onehot = onehot.astype(jnp.float32).reshape(b, q, 1, N_WAY)

    qt = QT
    assert q % qt == 0
    loss_sum = pl.pallas_call(
        functools.partial(_mn4_kernel, q=qt, c=c, hw=hw, m_s=m_s),
        grid=(b, q // qt),
        in_specs=[
            pl.BlockSpec((1, qt, c, hw), lambda i, j: (i, j, 0, 0)),
            pl.BlockSpec((1, c, nsp), lambda i, j: (i, 0, 0)),
            pl.BlockSpec((1, qt, 1, N_WAY), lambda i, j: (i, j, 0, 0)),
        ],
        out_specs=pl.BlockSpec((1, 1), lambda i, j: (0, 0)),
        out_shape=jax.ShapeDtypeStruct((1, 1), jnp.float32),
    )(qnat, smat, onehot)

    return loss_sum[0, 0] / (b * q)
